# baseline (speedup 1.0000x reference)
import functools

import jax
import jax.numpy as jnp
from jax import lax
from jax.experimental import pallas as pl
from jax.experimental.pallas import tpu as pltpu
from jax.experimental.pallas import tpu_sc as plsc

_B, _S, _H = 4, 2048, 4096
_NW, _KE, _ST = 512, 4, 3
_V, _D, _INTER = 100000, 100, 512
_DP = 128
_EPS = 1e-6


def _sc_gather(table, idx_flat):
    n = idx_flat.shape[0]
    nwork = 32
    per = n // nwork
    ch = 128
    mesh = plsc.VectorSubcoreMesh(core_axis_name="c", subcore_axis_name="s")

    @functools.partial(
        pl.kernel,
        out_type=jax.ShapeDtypeStruct((n, _DP), jnp.float32),
        mesh=mesh,
        scratch_types=[
            pltpu.VMEM((ch,), jnp.int32),
            pltpu.VMEM((ch,), jnp.int32),
            pltpu.VMEM((ch, _DP), jnp.float32),
            pltpu.VMEM((ch, _DP), jnp.float32),
            pltpu.SemaphoreType.DMA,
        ],
    )
    def k(table_hbm, idx_hbm, out_hbm, i0, i1, r0, r1, sem):
        wid = lax.axis_index("s") * 2 + lax.axis_index("c")
        base = wid * per
        pltpu.sync_copy(idx_hbm.at[pl.ds(base, ch)], i0)
        pltpu.sync_copy(idx_hbm.at[pl.ds(base + ch, ch)], i1)
        c0 = pltpu.async_copy(table_hbm.at[i0], r0, sem)
        c1 = pltpu.async_copy(table_hbm.at[i1], r1, sem)
        c0.wait()
        c1.wait()
        pltpu.sync_copy(r0, out_hbm.at[pl.ds(base, ch)])
        pltpu.sync_copy(r1, out_hbm.at[pl.ds(base + ch, ch)])

    return k(table, idx_flat)


_PV = 2000


def _pad_body(t_ref, o_ref):
    o_ref[...] = jnp.concatenate(
        [t_ref[...], jnp.zeros((_PV, _DP - _D), jnp.float32)], axis=1)


def _pad_call(table):
    return pl.pallas_call(
        _pad_body,
        grid=(_V // _PV,),
        in_specs=[pl.BlockSpec((_PV, _D), lambda i: (i, 0))],
        out_specs=pl.BlockSpec((_PV, _DP), lambda i: (i, 0)),
        out_shape=jax.ShapeDtypeStruct((_V, _DP), jnp.float32),
        compiler_params=pltpu.CompilerParams(
            dimension_semantics=("parallel",)),
    )(table)


_BSP = 1024


def _p_body(x_ref, ln_ref, w_ref, p_ref, xbf_ref):
    xb = x_ref[0]
    ss = jnp.sum(xb * xb, axis=1, keepdims=True)
    r = lax.rsqrt(ss / _H + _EPS)
    xln = (xb * ln_ref[0] * r).astype(jnp.bfloat16)
    xbf_ref[0] = xln
    p_ref[0] = lax.dot_general(
        xln, w_ref[...], (((1,), (0,)), ((), ())),
        preferred_element_type=jnp.float32).astype(jnp.bfloat16)


def _p_call(x, ln_row, down_bf):
    return pl.pallas_call(
        _p_body,
        grid=(_B, _S // _BSP),
        in_specs=[
            pl.BlockSpec((1, _BSP, _H), lambda b, s: (b, s, 0)),
            pl.BlockSpec((1, 1, _H), lambda b, s: (0, 0, 0)),
            pl.BlockSpec((_H, _INTER), lambda b, s: (0, 0)),
        ],
        out_specs=[
            pl.BlockSpec((1, _BSP, _INTER), lambda b, s: (b, s, 0)),
            pl.BlockSpec((1, _BSP, _H), lambda b, s: (b, s, 0)),
        ],
        out_shape=[
            jax.ShapeDtypeStruct((_B, _S, _INTER), jnp.bfloat16),
            jax.ShapeDtypeStruct((_B, _S, _H), jnp.bfloat16),
        ],
        compiler_params=pltpu.CompilerParams(
            dimension_semantics=("parallel", "parallel"),
            vmem_limit_bytes=100 * 1024 * 1024),
    )(x, ln_row, down_bf)


def _mid_body(p_ref, e_ref, s_ref, g_ref, u_ref, wc_ref, kg_ref):
    p2 = p_ref[0]

    lane_s = lax.broadcasted_iota(jnp.int32, (_NW, _S), 1)
    a = jnp.zeros((_NW, _S), jnp.float32)
    for t in range(_ST):
        wt = wc_ref[0][:, t:t + 1]
        a += (wt == lane_s).astype(jnp.float32)
    c3 = lax.dot_general(a.astype(jnp.bfloat16), p2,
                         (((1,), (0,)), ((), ())),
                         preferred_element_type=jnp.float32) * (1.0 / _ST)

    srow = s_ref[0:1]

    def silu_gate_up(e):
        eb = e.astype(jnp.bfloat16)
        g = lax.dot_general(eb, g_ref[...], (((1,), (1,)), ((), ())),
                            preferred_element_type=jnp.float32)
        u = lax.dot_general(eb, u_ref[...], (((1,), (1,)), ((), ())),
                            preferred_element_type=jnp.float32)
        return g * (1.0 / (1.0 + jnp.exp(-g))) * u

    es, aw = [], []
    for k in range(_KE):
        ek = e_ref[0, k]
        es.append(ek)
        sg = silu_gate_up(ek)
        aw.append(jnp.sum(sg * c3, axis=1, keepdims=True))
    sgs = silu_gate_up(srow)
    aw_s = jnp.sum(sgs * c3, axis=1, keepdims=True)

    m = aw_s
    for v in aw:
        m = jnp.maximum(m, v)
    pe = [jnp.exp(v - m) for v in aw]
    pss = jnp.exp(aw_s - m)
    den = pss
    for v in pe:
        den = den + v
    ctx = (pss / den) * srow
    for k in range(_KE):
        ctx = ctx + (pe[k] / den) * es[k]

    n_io = lax.broadcasted_iota(jnp.int32, (_NW, _S), 0)
    win_row = jnp.max(jnp.where(a > 0, n_io, -1), axis=0,
                      keepdims=True)
    oh_t = jnp.where((n_io == win_row) & (win_row >= 0),
                     1.0, 0.0).astype(jnp.bfloat16)
    kg_ref[0] = lax.dot_general(oh_t, ctx.astype(jnp.bfloat16),
                                (((0,), (0,)), ((), ())),
                                preferred_element_type=jnp.float32
                                ).astype(jnp.bfloat16)


def _mid_call(p, ents4, sent_rows, gate_p, up_p, words_subtoken):
    return pl.pallas_call(
        _mid_body,
        grid=(_B,),
        in_specs=[
            pl.BlockSpec((1, _S, _INTER), lambda b: (b, 0, 0)),
            pl.BlockSpec((1, _KE, _NW, _DP), lambda b: (b, 0, 0, 0)),
            pl.BlockSpec((8, _DP), lambda b: (0, 0)),
            pl.BlockSpec((_INTER, _DP), lambda b: (0, 0)),
            pl.BlockSpec((_INTER, _DP), lambda b: (0, 0)),
            pl.BlockSpec((1, _NW, _ST), lambda b: (b, 0, 0)),
        ],
        out_specs=pl.BlockSpec((1, _S, _DP), lambda b: (b, 0, 0)),
        out_shape=jax.ShapeDtypeStruct((_B, _S, _DP), jnp.bfloat16),
        compiler_params=pltpu.CompilerParams(
            dimension_semantics=("parallel",)),
    )(p, ents4, sent_rows, gate_p, up_p, words_subtoken)


_NR = _B * _S
_BS, _BO = 2048, 512
_SI, _OI = _NR // _BS, _H // _BO


def _final_body(xbf_ref, w1_ref, kg_ref, w2_ref, res_ref, b_ref,
                a_ref, o_ref):
    z = lax.dot_general(xbf_ref[0], w1_ref[...], (((1,), (1,)), ((), ())),
                        preferred_element_type=jnp.float32)
    z += lax.dot_general(kg_ref[0], w2_ref[...], (((1,), (1,)), ((), ())),
                         preferred_element_type=jnp.float32)
    z += b_ref[0]
    o_ref[0] = res_ref[0] + a_ref[0, 0] * (z * (1.0 / (1.0 + jnp.exp(-z))))


def _final_call(xbf, w1_bf, kg, w2_bf, res, bias_row, alpha2):
    out = pl.pallas_call(
        _final_body,
        grid=(_SI, _OI),
        in_specs=[
            pl.BlockSpec((1, _BS, _H), lambda s, o: (0, s, 0)),
            pl.BlockSpec((_BO, _H), lambda s, o: (o, 0)),
            pl.BlockSpec((1, _BS, _DP), lambda s, o: (0, s, 0)),
            pl.BlockSpec((_BO, _DP), lambda s, o: (o, 0)),
            pl.BlockSpec((1, _BS, _BO), lambda s, o: (0, s, o)),
            pl.BlockSpec((1, 1, _BO), lambda s, o: (0, 0, o)),
            pl.BlockSpec(memory_space=pltpu.MemorySpace.SMEM),
        ],
        out_specs=pl.BlockSpec((1, _BS, _BO), lambda s, o: (0, s, o)),
        out_shape=jax.ShapeDtypeStruct((1, _NR, _H), jnp.float32),
        compiler_params=pltpu.CompilerParams(
            dimension_semantics=("parallel", "parallel"),
            vmem_limit_bytes=120 * 1024 * 1024),
    )(xbf.reshape(1, _NR, _H), w1_bf, kg.reshape(1, _NR, _DP),
      w2_bf, res.reshape(1, _NR, _H), bias_row, alpha2)
    return out.reshape(_B, _S, _H)


def kernel(output_hidden_states, words_ents, words_subtoken, input_ids,
           concept_table, sentinel_w, ln_w, gate_w, up_w, down_w, mlp_w,
           mlp_b, alpha):
    x = output_hidden_states

    idx_flat = jnp.transpose(words_ents, (0, 2, 1)).reshape(-1)
    table_p = _pad_call(concept_table)
    ents = _sc_gather(table_p, idx_flat)
    ents4 = ents.reshape(_B, _KE, _NW, _DP)

    sent_rows = jnp.broadcast_to(
        jnp.pad(sentinel_w, ((0, 0), (0, _DP - _D))), (8, _DP))
    gate_p = jnp.pad(gate_w, ((0, 0), (0, _DP - _D))).astype(jnp.bfloat16)
    up_p = jnp.pad(up_w, ((0, 0), (0, _DP - _D))).astype(jnp.bfloat16)
    ln_row = ln_w.reshape(1, 1, _H)
    down_bf = down_w.astype(jnp.bfloat16)
    w1_bf = mlp_w[:, :_H].astype(jnp.bfloat16)
    w2_bf = jnp.pad(mlp_w[:, _H:], ((0, 0), (0, _DP - _D))).astype(jnp.bfloat16)
    bias_row = mlp_b.reshape(1, 1, _H)
    alpha2 = alpha.reshape(1, 1)

    p, xbf = _p_call(x, ln_row, down_bf)
    kg = _mid_call(p, ents4, sent_rows, gate_p, up_p, words_subtoken)
    return _final_call(xbf, w1_bf, kg, w2_bf, x, bias_row, alpha2)


# speedup vs baseline: 1.0006x; 1.0006x over previous
import functools

import jax
import jax.numpy as jnp
from jax import lax
from jax.experimental import pallas as pl
from jax.experimental.pallas import tpu as pltpu
from jax.experimental.pallas import tpu_sc as plsc

_B, _S, _H = 4, 2048, 4096
_NW, _KE, _ST = 512, 4, 3
_V, _D, _INTER = 100000, 100, 512
_DP = 128
_EPS = 1e-6


def _sc_gather(table, idx_flat):
    n = idx_flat.shape[0]
    nwork = 32
    per = n // nwork
    ch = 128
    mesh = plsc.VectorSubcoreMesh(core_axis_name="c", subcore_axis_name="s")

    @functools.partial(
        pl.kernel,
        out_type=jax.ShapeDtypeStruct((n, _DP), jnp.float32),
        mesh=mesh,
        scratch_types=[
            pltpu.VMEM((ch,), jnp.int32),
            pltpu.VMEM((ch,), jnp.int32),
            pltpu.VMEM((ch, _DP), jnp.float32),
            pltpu.VMEM((ch, _DP), jnp.float32),
            pltpu.SemaphoreType.DMA,
        ],
    )
    def k(table_hbm, idx_hbm, out_hbm, i0, i1, r0, r1, sem):
        wid = lax.axis_index("s") * 2 + lax.axis_index("c")
        base = wid * per
        pltpu.sync_copy(idx_hbm.at[pl.ds(base, ch)], i0)
        pltpu.sync_copy(idx_hbm.at[pl.ds(base + ch, ch)], i1)
        c0 = pltpu.async_copy(table_hbm.at[i0], r0, sem)
        c1 = pltpu.async_copy(table_hbm.at[i1], r1, sem)
        c0.wait()
        c1.wait()
        pltpu.sync_copy(r0, out_hbm.at[pl.ds(base, ch)])
        pltpu.sync_copy(r1, out_hbm.at[pl.ds(base + ch, ch)])

    return k(table, idx_flat)


_PV = 2000


def _pad_body(t_ref, o_ref):
    o_ref[...] = jnp.concatenate(
        [t_ref[...], jnp.zeros((_PV, _DP - _D), jnp.float32)], axis=1)


def _pad_call(table):
    return pl.pallas_call(
        _pad_body,
        grid=(_V // _PV,),
        in_specs=[pl.BlockSpec((_PV, _D), lambda i: (i, 0))],
        out_specs=pl.BlockSpec((_PV, _DP), lambda i: (i, 0)),
        out_shape=jax.ShapeDtypeStruct((_V, _DP), jnp.float32),
        compiler_params=pltpu.CompilerParams(
            dimension_semantics=("parallel",)),
    )(table)


_BSP = 512


def _p_body(x_ref, ln_ref, w_ref, p_ref, xbf_ref):
    xb = x_ref[0]
    ss = jnp.sum(xb * xb, axis=1, keepdims=True)
    r = lax.rsqrt(ss / _H + _EPS)
    xln = (xb * ln_ref[0] * r).astype(jnp.bfloat16)
    xbf_ref[0] = xln
    p_ref[0] = lax.dot_general(
        xln, w_ref[...], (((1,), (0,)), ((), ())),
        preferred_element_type=jnp.float32).astype(jnp.bfloat16)


def _p_call(x, ln_row, down_bf):
    return pl.pallas_call(
        _p_body,
        grid=(_B, _S // _BSP),
        in_specs=[
            pl.BlockSpec((1, _BSP, _H), lambda b, s: (b, s, 0)),
            pl.BlockSpec((1, 1, _H), lambda b, s: (0, 0, 0)),
            pl.BlockSpec((_H, _INTER), lambda b, s: (0, 0)),
        ],
        out_specs=[
            pl.BlockSpec((1, _BSP, _INTER), lambda b, s: (b, s, 0)),
            pl.BlockSpec((1, _BSP, _H), lambda b, s: (b, s, 0)),
        ],
        out_shape=[
            jax.ShapeDtypeStruct((_B, _S, _INTER), jnp.bfloat16),
            jax.ShapeDtypeStruct((_B, _S, _H), jnp.bfloat16),
        ],
        compiler_params=pltpu.CompilerParams(
            dimension_semantics=("parallel", "parallel")),
    )(x, ln_row, down_bf)


def _mid_body(p_ref, e_ref, s_ref, g_ref, u_ref, wc_ref, kg_ref):
    p2 = p_ref[0]

    lane_s = lax.broadcasted_iota(jnp.int32, (_NW, _S), 1)
    a = jnp.zeros((_NW, _S), jnp.float32)
    for t in range(_ST):
        wt = wc_ref[0][:, t:t + 1]
        a += (wt == lane_s).astype(jnp.float32)
    c3 = lax.dot_general(a.astype(jnp.bfloat16), p2,
                         (((1,), (0,)), ((), ())),
                         preferred_element_type=jnp.float32) * (1.0 / _ST)

    srow = s_ref[0:1]

    def silu_gate_up(e):
        eb = e.astype(jnp.bfloat16)
        g = lax.dot_general(eb, g_ref[...], (((1,), (1,)), ((), ())),
                            preferred_element_type=jnp.float32)
        u = lax.dot_general(eb, u_ref[...], (((1,), (1,)), ((), ())),
                            preferred_element_type=jnp.float32)
        return g * (1.0 / (1.0 + jnp.exp(-g))) * u

    es, aw = [], []
    for k in range(_KE):
        ek = e_ref[0, k]
        es.append(ek)
        sg = silu_gate_up(ek)
        aw.append(jnp.sum(sg * c3, axis=1, keepdims=True))
    sgs = silu_gate_up(srow)
    aw_s = jnp.sum(sgs * c3, axis=1, keepdims=True)

    m = aw_s
    for v in aw:
        m = jnp.maximum(m, v)
    pe = [jnp.exp(v - m) for v in aw]
    pss = jnp.exp(aw_s - m)
    den = pss
    for v in pe:
        den = den + v
    ctx = (pss / den) * srow
    for k in range(_KE):
        ctx = ctx + (pe[k] / den) * es[k]

    n_io = lax.broadcasted_iota(jnp.int32, (_NW, _S), 0)
    win_row = jnp.max(jnp.where(a > 0, n_io, -1), axis=0,
                      keepdims=True)
    oh_t = jnp.where((n_io == win_row) & (win_row >= 0),
                     1.0, 0.0).astype(jnp.bfloat16)
    kg_ref[0] = lax.dot_general(oh_t, ctx.astype(jnp.bfloat16),
                                (((0,), (0,)), ((), ())),
                                preferred_element_type=jnp.float32
                                ).astype(jnp.bfloat16)


def _mid_call(p, ents4, sent_rows, gate_p, up_p, words_subtoken):
    return pl.pallas_call(
        _mid_body,
        grid=(_B,),
        in_specs=[
            pl.BlockSpec((1, _S, _INTER), lambda b: (b, 0, 0)),
            pl.BlockSpec((1, _KE, _NW, _DP), lambda b: (b, 0, 0, 0)),
            pl.BlockSpec((8, _DP), lambda b: (0, 0)),
            pl.BlockSpec((_INTER, _DP), lambda b: (0, 0)),
            pl.BlockSpec((_INTER, _DP), lambda b: (0, 0)),
            pl.BlockSpec((1, _NW, _ST), lambda b: (b, 0, 0)),
        ],
        out_specs=pl.BlockSpec((1, _S, _DP), lambda b: (b, 0, 0)),
        out_shape=jax.ShapeDtypeStruct((_B, _S, _DP), jnp.bfloat16),
        compiler_params=pltpu.CompilerParams(
            dimension_semantics=("parallel",)),
    )(p, ents4, sent_rows, gate_p, up_p, words_subtoken)


_NR = _B * _S
_BS, _BO = 2048, 512
_SI, _OI = _NR // _BS, _H // _BO


def _final_body(xbf_ref, w1_ref, kg_ref, w2_ref, res_ref, b_ref,
                a_ref, o_ref):
    z = lax.dot_general(xbf_ref[0], w1_ref[...], (((1,), (1,)), ((), ())),
                        preferred_element_type=jnp.float32)
    z += lax.dot_general(kg_ref[0], w2_ref[...], (((1,), (1,)), ((), ())),
                         preferred_element_type=jnp.float32)
    z += b_ref[0]
    o_ref[0] = res_ref[0] + a_ref[0, 0] * (z * (1.0 / (1.0 + jnp.exp(-z))))


def _final_call(xbf, w1_bf, kg, w2_bf, res, bias_row, alpha2):
    out = pl.pallas_call(
        _final_body,
        grid=(_SI, _OI),
        in_specs=[
            pl.BlockSpec((1, _BS, _H), lambda s, o: (0, s, 0)),
            pl.BlockSpec((_BO, _H), lambda s, o: (o, 0)),
            pl.BlockSpec((1, _BS, _DP), lambda s, o: (0, s, 0)),
            pl.BlockSpec((_BO, _DP), lambda s, o: (o, 0)),
            pl.BlockSpec((1, _BS, _BO), lambda s, o: (0, s, o)),
            pl.BlockSpec((1, 1, _BO), lambda s, o: (0, 0, o)),
            pl.BlockSpec(memory_space=pltpu.MemorySpace.SMEM),
        ],
        out_specs=pl.BlockSpec((1, _BS, _BO), lambda s, o: (0, s, o)),
        out_shape=jax.ShapeDtypeStruct((1, _NR, _H), jnp.float32),
        compiler_params=pltpu.CompilerParams(
            dimension_semantics=("parallel", "parallel"),
            vmem_limit_bytes=120 * 1024 * 1024),
    )(xbf.reshape(1, _NR, _H), w1_bf, kg.reshape(1, _NR, _DP),
      w2_bf, res.reshape(1, _NR, _H), bias_row, alpha2)
    return out.reshape(_B, _S, _H)


def kernel(output_hidden_states, words_ents, words_subtoken, input_ids,
           concept_table, sentinel_w, ln_w, gate_w, up_w, down_w, mlp_w,
           mlp_b, alpha):
    x = output_hidden_states

    idx_flat = jnp.transpose(words_ents, (0, 2, 1)).reshape(-1)
    table_p = _pad_call(concept_table)
    ents = _sc_gather(table_p, idx_flat)
    ents4 = ents.reshape(_B, _KE, _NW, _DP)

    sent_rows = jnp.broadcast_to(
        jnp.pad(sentinel_w, ((0, 0), (0, _DP - _D))), (8, _DP))
    gate_p = jnp.pad(gate_w, ((0, 0), (0, _DP - _D))).astype(jnp.bfloat16)
    up_p = jnp.pad(up_w, ((0, 0), (0, _DP - _D))).astype(jnp.bfloat16)
    ln_row = ln_w.reshape(1, 1, _H)
    down_bf = down_w.astype(jnp.bfloat16)
    w1_bf = mlp_w[:, :_H].astype(jnp.bfloat16)
    w2_bf = jnp.pad(mlp_w[:, _H:], ((0, 0), (0, _DP - _D))).astype(jnp.bfloat16)
    bias_row = mlp_b.reshape(1, 1, _H)
    alpha2 = alpha.reshape(1, 1)

    p, xbf = _p_call(x, ln_row, down_bf)
    kg = _mid_call(p, ents4, sent_rows, gate_p, up_p, words_subtoken)
    return _final_call(xbf, w1_bf, kg, w2_bf, x, bias_row, alpha2)
